# baseline (speedup 1.0000x reference)
import functools

import jax
import jax.numpy as jnp
from jax import lax
from jax.experimental import pallas as pl
from jax.experimental.pallas import tpu as pltpu
from jax.experimental.pallas import tpu_sc as plsc

_INFO = plsc.get_sparse_core_info()
_NC = _INFO.num_cores
_NS = _INFO.num_subcores
_NW = _NC * _NS

_CHUNK = 64
_NBUF = 2
_LEAD = 1


def _gather_rows(B, D):
    b_per_w = B // _NW
    n = b_per_w // _CHUNK
    assert n % _NBUF == 0 and n >= _NBUF + _LEAD
    mesh = plsc.VectorSubcoreMesh(core_axis_name="c", subcore_axis_name="s")

    @functools.partial(
        pl.kernel,
        out_type=jax.ShapeDtypeStruct((B, D), jnp.float32),
        mesh=mesh,
        scratch_types=[
            pltpu.VMEM((b_per_w,), jnp.int32),
            pltpu.VMEM((_NBUF, _CHUNK, D), jnp.float32),
        ]
        + [pltpu.SemaphoreType.DMA] * (2 * _NBUF),
    )
    def run(W_hbm, idx_hbm, out_hbm, idx_v, rows_v, *sems):
        gsem, wsem = sems[:_NBUF], sems[_NBUF:]
        wid = lax.axis_index("s") * _NC + lax.axis_index("c")
        base = wid * b_per_w
        pltpu.sync_copy(idx_hbm.at[pl.ds(base, b_per_w)], idx_v)

        def start_gather(c, b):
            pltpu.async_copy(
                W_hbm.at[idx_v.at[pl.ds(c * _CHUNK, _CHUNK)]],
                rows_v.at[b], gsem[b],
            )

        def wait_gather(b):
            pltpu.make_async_copy(
                W_hbm.at[pl.ds(0, _CHUNK)], rows_v.at[b], gsem[b]
            ).wait()

        def start_write(c, b):
            pltpu.async_copy(
                rows_v.at[b], out_hbm.at[pl.ds(base + c * _CHUNK, _CHUNK)],
                wsem[b],
            )

        def wait_write(b):
            pltpu.make_async_copy(
                rows_v.at[b], out_hbm.at[pl.ds(base, _CHUNK)], wsem[b]
            ).wait()

        for c in range(_LEAD):
            start_gather(c, c % _NBUF)

        for j in range(_NBUF):
            wait_gather(j)
            start_write(j, j)
            c3, b3 = j + _LEAD, (j + _LEAD) % _NBUF
            if c3 >= _NBUF:
                wait_write(b3)
            start_gather(c3, b3)

        @pl.loop(_NBUF, n, step=_NBUF)
        def _(i):
            for j in range(_NBUF):
                c = i + j
                wait_gather(j)
                start_write(c, j)
                c3, b3 = c + _LEAD, (j + _LEAD) % _NBUF

                @pl.when(c3 < n)
                def _():
                    wait_write(b3)
                    start_gather(c3, b3)

        for b in range(_NBUF):
            wait_write(b)

    return run


@jax.jit
def kernel(t, W):
    B = t.shape[0] * t.shape[1]
    D = W.shape[1]
    idx = t.reshape(B).astype(jnp.int32)
    out = _gather_rows(B, D)(W, idx)
    return out.reshape(t.shape[0], t.shape[1], D)


# speedup vs baseline: 1.0319x; 1.0319x over previous
import functools

import jax
import jax.numpy as jnp
from jax import lax
from jax.experimental import pallas as pl
from jax.experimental.pallas import tpu as pltpu
from jax.experimental.pallas import tpu_sc as plsc

_INFO = plsc.get_sparse_core_info()
_NC = _INFO.num_cores
_NS = _INFO.num_subcores
_NW = _NC * _NS

_CHUNK = 16
_NBUF = 8
_LEAD = 4


def _gather_rows(B, D):
    b_per_w = B // _NW
    n = b_per_w // _CHUNK
    assert n % _NBUF == 0 and n >= _NBUF + _LEAD
    mesh = plsc.VectorSubcoreMesh(core_axis_name="c", subcore_axis_name="s")

    @functools.partial(
        pl.kernel,
        out_type=jax.ShapeDtypeStruct((B, D), jnp.float32),
        mesh=mesh,
        scratch_types=[
            pltpu.VMEM((b_per_w,), jnp.int32),
            pltpu.VMEM((_NBUF, _CHUNK, D), jnp.float32),
        ]
        + [pltpu.SemaphoreType.DMA] * (2 * _NBUF),
    )
    def run(W_hbm, idx_hbm, out_hbm, idx_v, rows_v, *sems):
        gsem, wsem = sems[:_NBUF], sems[_NBUF:]
        wid = lax.axis_index("s") * _NC + lax.axis_index("c")
        base = wid * b_per_w
        w_per_row = idx_hbm.shape[1] // b_per_w
        pltpu.sync_copy(
            idx_hbm.at[wid // w_per_row,
                       pl.ds((wid % w_per_row) * b_per_w, b_per_w)],
            idx_v,
        )

        def start_gather(c, b):
            pltpu.async_copy(
                W_hbm.at[idx_v.at[pl.ds(c * _CHUNK, _CHUNK)]],
                rows_v.at[b], gsem[b],
            )

        def wait_gather(b):
            pltpu.make_async_copy(
                W_hbm.at[pl.ds(0, _CHUNK)], rows_v.at[b], gsem[b]
            ).wait()

        def start_write(c, b):
            pltpu.async_copy(
                rows_v.at[b], out_hbm.at[pl.ds(base + c * _CHUNK, _CHUNK)],
                wsem[b],
            )

        def wait_write(b):
            pltpu.make_async_copy(
                rows_v.at[b], out_hbm.at[pl.ds(base, _CHUNK)], wsem[b]
            ).wait()

        for c in range(_LEAD):
            start_gather(c, c % _NBUF)

        for j in range(_NBUF):
            wait_gather(j)
            start_write(j, j)
            c3, b3 = j + _LEAD, (j + _LEAD) % _NBUF
            if c3 >= _NBUF:
                wait_write(b3)
            start_gather(c3, b3)

        @pl.loop(_NBUF, n, step=_NBUF)
        def _(i):
            for j in range(_NBUF):
                c = i + j
                wait_gather(j)
                start_write(c, j)
                c3, b3 = c + _LEAD, (j + _LEAD) % _NBUF

                @pl.when(c3 < n)
                def _():
                    wait_write(b3)
                    start_gather(c3, b3)

        for b in range(_NBUF):
            wait_write(b)

    return run


@jax.jit
def kernel(t, W):
    B = t.shape[0] * t.shape[1]
    D = W.shape[1]
    out = _gather_rows(B, D)(W, t)
    return out.reshape(t.shape[0], t.shape[1], D)


# speedup vs baseline: 1.0348x; 1.0028x over previous
import functools

import jax
import jax.numpy as jnp
from jax import lax
from jax.experimental import pallas as pl
from jax.experimental.pallas import tpu as pltpu
from jax.experimental.pallas import tpu_sc as plsc

_INFO = plsc.get_sparse_core_info()
_NC = _INFO.num_cores
_NS = _INFO.num_subcores
_NW = _NC * _NS

_CHUNK = 16
_NBUF = 8
_LEAD = 4


def _gather_rows(B, D):
    b_per_w = B // _NW
    n = b_per_w // _CHUNK
    assert n % _NBUF == 0 and n >= _NBUF + _LEAD
    mesh = plsc.VectorSubcoreMesh(core_axis_name="c", subcore_axis_name="s")

    @functools.partial(
        pl.kernel,
        out_type=jax.ShapeDtypeStruct((B, D), jnp.float32),
        mesh=mesh,
        scratch_types=[
            pltpu.VMEM((b_per_w,), jnp.int32),
            pltpu.VMEM((_NBUF, _CHUNK, D), jnp.float32),
        ]
        + [pltpu.SemaphoreType.DMA] * (2 * _NBUF),
    )
    def run(W_hbm, idx_hbm, out_hbm, idx_v, rows_v, *sems):
        gsem, wsem = sems[:_NBUF], sems[_NBUF:]
        wid = lax.axis_index("s") * _NC + lax.axis_index("c")
        base = wid * b_per_w
        w_per_row = idx_hbm.shape[1] // b_per_w
        pltpu.sync_copy(
            idx_hbm.at[wid // w_per_row,
                       pl.ds((wid % w_per_row) * b_per_w, b_per_w)],
            idx_v,
        )

        def start_gather(c, b):
            pltpu.async_copy(
                W_hbm.at[idx_v.at[pl.ds(c * _CHUNK, _CHUNK)]],
                rows_v.at[b], gsem[b],
            )

        def wait_gather(b):
            pltpu.make_async_copy(
                W_hbm.at[pl.ds(0, _CHUNK)], rows_v.at[b], gsem[b]
            ).wait()

        def start_write(c, b):
            pltpu.async_copy(
                rows_v.at[b], out_hbm.at[pl.ds(base + c * _CHUNK, _CHUNK)],
                wsem[b],
            )

        def wait_write(b):
            pltpu.make_async_copy(
                rows_v.at[b], out_hbm.at[pl.ds(base, _CHUNK)], wsem[b]
            ).wait()

        for c in range(_LEAD):
            start_gather(c, c % _NBUF)

        for j in range(_NBUF):
            wait_gather(j)
            start_write(j, j)
            c3, b3 = j + _LEAD, (j + _LEAD) % _NBUF
            if c3 >= _NBUF:
                wait_write(b3)
            start_gather(c3, b3)

        @pl.loop(_NBUF, n, step=_NBUF)
        def _(i):
            for j in range(_NBUF):
                c = i + j
                wait_gather(j)
                start_write(c, j)
                c3, b3 = c + _LEAD, (j + _LEAD) % _NBUF

                @pl.when(c3 < n)
                def _():
                    wait_write(b3)
                    start_gather(c3, b3)

        for b in range(_NBUF):
            wait_write(b)

    return run


@jax.jit
def kernel(t, W):
    B = t.shape[0] * t.shape[1]
    D = W.shape[1]
    out = _gather_rows(B, D)(W, t.astype(jnp.int32))
    return out.reshape(t.shape[0], t.shape[1], D)
